# baseline (speedup 1.0000x reference)
import functools
import math

import jax
import jax.numpy as jnp
from jax import lax
from jax.experimental import pallas as pl
from jax.experimental.pallas import tpu as pltpu
from jax.experimental.pallas import tpu_sc as plsc

D = 64
SCALE = math.sqrt(D)
LANES = 16
NC, NS = 2, 16
NW = NC * NS
CHUNK = 128
NBUF = 4


@functools.partial(jax.jit, static_argnums=(2, 3))
def _embed(idx, table, nch, b):
    bpw = nch * CHUNK
    mesh = plsc.VectorSubcoreMesh(core_axis_name="c", subcore_axis_name="s")

    @functools.partial(
        pl.kernel,
        out_type=jax.ShapeDtypeStruct((b, D), jnp.float32),
        mesh=mesh,
        scratch_types=[
            pltpu.VMEM((nch, CHUNK), jnp.int32),
            pltpu.VMEM((NBUF, CHUNK, D), jnp.float32),
            pltpu.SemaphoreType.DMA((NBUF,)),
            pltpu.SemaphoreType.DMA((NBUF,)),
        ],
        compiler_params=pltpu.CompilerParams(use_tc_tiling_on_sc=False),
    )
    def run(idx_hbm, table_hbm, out_hbm, idx_v, rows_v, gsem, osem):
        wid = lax.axis_index("s") * NC + lax.axis_index("c")
        base = wid * bpw
        pltpu.sync_copy(idx_hbm.at[wid], idx_v)

        def gather(g, slot):
            return pltpu.make_async_copy(
                table_hbm.at[idx_v.at[g]], rows_v.at[slot], gsem.at[slot])

        def put(g, slot):
            return pltpu.make_async_copy(
                rows_v.at[slot],
                out_hbm.at[pl.ds(base + g * CHUNK, CHUNK)],
                osem.at[slot])

        for slot in range(NBUF):
            gather(slot, slot).start()

        @pl.loop(0, nch, step=NBUF)
        def _outer(g0):
            for slot in range(NBUF):
                g = g0 + slot
                gather(g, slot).wait()

                @pl.loop(0, CHUNK)
                def _scale(r):
                    for c in range(D // LANES):
                        sl = pl.ds(c * LANES, LANES)
                        rows_v[slot, r, sl] = rows_v[slot, r, sl] * SCALE

                put(g, slot).start()
                nxt = g + NBUF

                @pl.when(nxt < nch)
                def _():
                    put(g, slot).wait()
                    gather(nxt, slot).start()

        for slot in range(NBUF):
            put(nch - NBUF + slot, slot).wait()

    return run(idx, table)


def kernel(x, table):
    b = x.size
    nch = b // (NW * CHUNK)
    idx = x.reshape(NW, nch, CHUNK).astype(jnp.int32)
    out = _embed(idx, table, nch, b)
    return out.reshape(x.shape + (D,))
